# Initial kernel scaffold; baseline (speedup 1.0000x reference)
import functools

import jax
import jax.numpy as jnp
from jax import lax
from jax.experimental import pallas as pl
from jax.experimental.pallas import tpu as pltpu
from jax.experimental.pallas import tpu_sc as plsc

NUM_EMBEDDINGS = 100000
EMBEDDING_DIM = 128
OUT_DIM = 32

ROW_BLOCK = 2000


def _proj_body(table_ref, w_ref, b_ref, out_ref):
    acc = lax.dot_general(
        table_ref[...], w_ref[...],
        dimension_numbers=(((1,), (1,)), ((), ())),
        preferred_element_type=jnp.float32,
    )
    out_ref[...] = acc + b_ref[...]


def _project_table(table, W, b2d):
    grid = NUM_EMBEDDINGS // ROW_BLOCK
    return pl.pallas_call(
        _proj_body,
        grid=(grid,),
        in_specs=[
            pl.BlockSpec((ROW_BLOCK, EMBEDDING_DIM), lambda i: (i, 0)),
            pl.BlockSpec((OUT_DIM, EMBEDDING_DIM), lambda i: (0, 0)),
            pl.BlockSpec((1, OUT_DIM), lambda i: (0, 0)),
        ],
        out_specs=pl.BlockSpec((ROW_BLOCK, OUT_DIM), lambda i: (i, 0)),
        out_shape=jax.ShapeDtypeStruct((NUM_EMBEDDINGS, OUT_DIM), jnp.float32),
    )(table, W, b2d)


_INFO = plsc.get_sparse_core_info()
_NC = _INFO.num_cores
_NS = _INFO.num_subcores
_NW = _NC * _NS
_CHUNK = 128


def _make_gather(total):
    per_w = total // _NW
    n_chunks = per_w // _CHUNK
    mesh = plsc.VectorSubcoreMesh(core_axis_name="c", subcore_axis_name="s")

    @functools.partial(
        pl.kernel,
        mesh=mesh,
        out_type=jax.ShapeDtypeStruct((total, OUT_DIM), jnp.float32),
        scratch_types=[
            pltpu.VMEM((per_w,), jnp.int32),
            pltpu.VMEM((_CHUNK, OUT_DIM), jnp.float32),
            pltpu.SemaphoreType.DMA,
        ],
    )
    def gather_k(idx_hbm, proj_hbm, out_hbm, idx_v, rows_v, sem):
        wid = lax.axis_index("s") * _NC + lax.axis_index("c")
        base = wid * per_w
        pltpu.sync_copy(idx_hbm.at[pl.ds(base, per_w)], idx_v)

        def body(j, carry):
            pltpu.async_copy(
                proj_hbm.at[idx_v.at[pl.ds(j * _CHUNK, _CHUNK)]], rows_v, sem
            ).wait()
            pltpu.sync_copy(rows_v, out_hbm.at[pl.ds(base + j * _CHUNK, _CHUNK)])
            return carry

        lax.fori_loop(0, n_chunks, body, 0, unroll=False)

    return gather_k


def kernel(x, table, W, b):
    bsz, seq = x.shape
    total = bsz * seq
    proj = _project_table(table, W, b.reshape(1, OUT_DIM))
    flat_idx = x.reshape(total)
    out = _make_gather(total)(flat_idx, proj)
    return out.reshape(bsz, seq, OUT_DIM)


# speedup vs baseline: 2.0074x; 2.0074x over previous
import functools

import jax
import jax.numpy as jnp
from jax import lax
from jax.experimental import pallas as pl
from jax.experimental.pallas import tpu as pltpu
from jax.experimental.pallas import tpu_sc as plsc

NUM_EMBEDDINGS = 100000
EMBEDDING_DIM = 128
OUT_DIM = 32

ROW_BLOCK = 2000


def _proj_body(table_ref, w_ref, b_ref, out_ref):
    acc = lax.dot_general(
        table_ref[...], w_ref[...],
        dimension_numbers=(((1,), (1,)), ((), ())),
        preferred_element_type=jnp.float32,
    )
    out_ref[...] = acc + b_ref[...]


def _project_table(table, W, b2d):
    grid = NUM_EMBEDDINGS // ROW_BLOCK
    return pl.pallas_call(
        _proj_body,
        grid=(grid,),
        in_specs=[
            pl.BlockSpec((ROW_BLOCK, EMBEDDING_DIM), lambda i: (i, 0)),
            pl.BlockSpec((OUT_DIM, EMBEDDING_DIM), lambda i: (0, 0)),
            pl.BlockSpec((1, OUT_DIM), lambda i: (0, 0)),
        ],
        out_specs=pl.BlockSpec((ROW_BLOCK, OUT_DIM), lambda i: (i, 0)),
        out_shape=jax.ShapeDtypeStruct((NUM_EMBEDDINGS, OUT_DIM), jnp.float32),
    )(table, W, b2d)


_INFO = plsc.get_sparse_core_info()
_NC = _INFO.num_cores
_NS = _INFO.num_subcores
_NW = _NC * _NS
_CHUNK = 128


def _make_gather(total):
    per_w = total // _NW
    n_chunks = per_w // _CHUNK
    mesh = plsc.VectorSubcoreMesh(core_axis_name="c", subcore_axis_name="s")

    @functools.partial(
        pl.kernel,
        mesh=mesh,
        out_type=jax.ShapeDtypeStruct((total, OUT_DIM), jnp.float32),
        scratch_types=[
            pltpu.VMEM((per_w,), jnp.int32),
            pltpu.VMEM((_CHUNK, OUT_DIM), jnp.float32),
            pltpu.SemaphoreType.DMA,
        ],
        compiler_params=pltpu.CompilerParams(use_tc_tiling_on_sc=False),
    )
    def gather_k(idx_hbm, proj_hbm, out_hbm, idx_v, rows_v, sem):
        wid = lax.axis_index("s") * _NC + lax.axis_index("c")
        base = wid * per_w
        pltpu.sync_copy(idx_hbm.at[pl.ds(base, per_w)], idx_v)

        def body(j, carry):
            pltpu.async_copy(
                proj_hbm.at[idx_v.at[pl.ds(j * _CHUNK, _CHUNK)]], rows_v, sem
            ).wait()
            pltpu.sync_copy(rows_v, out_hbm.at[pl.ds(base + j * _CHUNK, _CHUNK)])
            return carry

        lax.fori_loop(0, n_chunks, body, 0, unroll=False)

    return gather_k


def kernel(x, table, W, b):
    bsz, seq = x.shape
    total = bsz * seq
    proj = _project_table(table, W, b.reshape(1, OUT_DIM))
    flat_idx = x.reshape(total)
    out = _make_gather(total)(flat_idx, proj)
    return out.reshape(bsz, seq, OUT_DIM)


# speedup vs baseline: 2.2759x; 1.1337x over previous
import functools

import jax
import jax.numpy as jnp
from jax import lax
from jax.experimental import pallas as pl
from jax.experimental.pallas import tpu as pltpu
from jax.experimental.pallas import tpu_sc as plsc

NUM_EMBEDDINGS = 100000
EMBEDDING_DIM = 128
OUT_DIM = 32

ROW_BLOCK = 4000


def _proj_body(table_ref, w_ref, b_ref, out_ref):
    t4 = table_ref[...].reshape(ROW_BLOCK // 4, 4, EMBEDDING_DIM)
    for k in range(4):
        acc = lax.dot_general(
            t4[:, k, :], w_ref[...],
            dimension_numbers=(((1,), (1,)), ((), ())),
            preferred_element_type=jnp.float32,
        )
        out_ref[:, k * OUT_DIM:(k + 1) * OUT_DIM] = acc + b_ref[...]


def _project_table(table, W, b2d):
    grid = NUM_EMBEDDINGS // ROW_BLOCK
    return pl.pallas_call(
        _proj_body,
        grid=(grid,),
        in_specs=[
            pl.BlockSpec((ROW_BLOCK, EMBEDDING_DIM), lambda i: (i, 0)),
            pl.BlockSpec((OUT_DIM, EMBEDDING_DIM), lambda i: (0, 0)),
            pl.BlockSpec((1, OUT_DIM), lambda i: (0, 0)),
        ],
        out_specs=pl.BlockSpec((ROW_BLOCK // 4, 4 * OUT_DIM), lambda i: (i, 0)),
        out_shape=jax.ShapeDtypeStruct((NUM_EMBEDDINGS // 4, 4 * OUT_DIM), jnp.float32),
    )(table, W, b2d)


_INFO = plsc.get_sparse_core_info()
_NC = _INFO.num_cores
_NS = _INFO.num_subcores
_NW = _NC * _NS
_CHUNK = 128


def _make_gather(total):
    per_w = total // _NW
    n_chunks = per_w // _CHUNK
    mesh = plsc.VectorSubcoreMesh(core_axis_name="c", subcore_axis_name="s")

    @functools.partial(
        pl.kernel,
        mesh=mesh,
        out_type=jax.ShapeDtypeStruct((total, OUT_DIM), jnp.float32),
        scratch_types=[
            pltpu.VMEM((per_w,), jnp.int32),
            pltpu.VMEM((_CHUNK, OUT_DIM), jnp.float32),
            pltpu.SemaphoreType.DMA,
        ],
        compiler_params=pltpu.CompilerParams(use_tc_tiling_on_sc=False),
    )
    def gather_k(idx_hbm, proj_hbm, out_hbm, idx_v, rows_v, sem):
        wid = lax.axis_index("s") * _NC + lax.axis_index("c")
        base = wid * per_w
        pltpu.sync_copy(idx_hbm.at[pl.ds(base, per_w)], idx_v)

        def body(j, carry):
            pltpu.async_copy(
                proj_hbm.at[idx_v.at[pl.ds(j * _CHUNK, _CHUNK)]], rows_v, sem
            ).wait()
            pltpu.sync_copy(rows_v, out_hbm.at[pl.ds(base + j * _CHUNK, _CHUNK)])
            return carry

        lax.fori_loop(0, n_chunks, body, 0, unroll=False)

    return gather_k


def kernel(x, table, W, b):
    bsz, seq = x.shape
    total = bsz * seq
    proj = _project_table(table, W, b.reshape(1, OUT_DIM))
    proj = proj.reshape(NUM_EMBEDDINGS, OUT_DIM)
    flat_idx = x.reshape(total)
    out = _make_gather(total)(flat_idx, proj)
    return out.reshape(bsz, seq, OUT_DIM)


# speedup vs baseline: 3.5154x; 1.5446x over previous
import functools

import jax
import jax.numpy as jnp
from jax import lax
from jax.experimental import pallas as pl
from jax.experimental.pallas import tpu as pltpu
from jax.experimental.pallas import tpu_sc as plsc

NUM_EMBEDDINGS = 100000
EMBEDDING_DIM = 128
OUT_DIM = 32

ROW_BLOCK = 4000


def _proj_body(table_ref, w_ref, b_ref, out_ref):
    t4 = table_ref[...].reshape(ROW_BLOCK // 4, 4, EMBEDDING_DIM)
    for k in range(4):
        acc = lax.dot_general(
            t4[:, k, :], w_ref[...],
            dimension_numbers=(((1,), (1,)), ((), ())),
            preferred_element_type=jnp.float32,
        )
        out_ref[:, k * OUT_DIM:(k + 1) * OUT_DIM] = acc + b_ref[...]


def _project_table(table, W, b2d):
    grid = NUM_EMBEDDINGS // ROW_BLOCK
    return pl.pallas_call(
        _proj_body,
        grid=(grid,),
        in_specs=[
            pl.BlockSpec((ROW_BLOCK, EMBEDDING_DIM), lambda i: (i, 0)),
            pl.BlockSpec((OUT_DIM, EMBEDDING_DIM), lambda i: (0, 0)),
            pl.BlockSpec((1, OUT_DIM), lambda i: (0, 0)),
        ],
        out_specs=pl.BlockSpec((ROW_BLOCK // 4, 4 * OUT_DIM), lambda i: (i, 0)),
        out_shape=jax.ShapeDtypeStruct((NUM_EMBEDDINGS // 4, 4 * OUT_DIM), jnp.float32),
    )(table, W, b2d)


_INFO = plsc.get_sparse_core_info()
_NC = _INFO.num_cores
_NS = _INFO.num_subcores
_NW = _NC * _NS

_B = 4096
_L = 50
_NB = _B // _NW


def _make_lookup():
    mesh = plsc.VectorSubcoreMesh(core_axis_name="c", subcore_axis_name="s")

    @functools.partial(
        pl.kernel,
        mesh=mesh,
        out_type=jax.ShapeDtypeStruct((_L * (OUT_DIM // 8) * (_B // 128) * 8 * 128,),
                                      jnp.float32),
        scratch_types=[
            pltpu.VMEM((_NB * _L,), jnp.int32),
            pltpu.VMEM((_NB,), jnp.int32),
            pltpu.VMEM((_NB, OUT_DIM), jnp.float32),
            pltpu.VMEM((4096,), jnp.float32),
            pltpu.SemaphoreType.DMA,
        ],
        compiler_params=pltpu.CompilerParams(
            use_tc_tiling_on_sc=False, needs_layout_passes=False
        ),
    )
    def lookup_k(idx_hbm, proj_hbm, out_hbm, idx_v, col_v, rows_v, tile_v, sem):
        wid = lax.axis_index("s") * _NC + lax.axis_index("c")
        iota16 = lax.iota(jnp.int32, 16)
        iota128 = iota16 * 128
        iota50 = iota16 * _L

        pltpu.sync_copy(idx_hbm.at[pl.ds(wid * (_NB * _L), _NB * _L)], idx_v)

        def lbody(l, carry):
            for k in range(8):
                vals = plsc.load_gather(idx_v, [iota50 + (16 * _L * k + l)])
                col_v[pl.ds(16 * k, 16)] = vals
            pltpu.async_copy(proj_hbm.at[col_v], rows_v, sem).wait()

            def bbody(bi, c2):
                v0 = rows_v[bi, pl.ds(0, 16)]
                v1 = rows_v[bi, pl.ds(16, 16)]
                plsc.store_scatter(tile_v, [iota128 + bi], v0)
                plsc.store_scatter(tile_v, [iota128 + (bi + 2048)], v1)
                return c2

            lax.fori_loop(0, _NB, bbody, 0, unroll=False)

            for ot in range(4):
                off = ((l * 4 + ot) * 32 + wid) * 1024
                pltpu.sync_copy(
                    tile_v.at[pl.ds(ot * 1024, 1024)],
                    out_hbm.at[pl.ds(off, 1024)],
                )
            return carry

        lax.fori_loop(0, _L, lbody, 0, unroll=False)

    return lookup_k


def kernel(x, table, W, b):
    proj = _project_table(table, W, b.reshape(1, OUT_DIM))
    proj = proj.reshape(NUM_EMBEDDINGS, OUT_DIM)
    out1d = _make_lookup()(x.reshape(_B * _L), proj)
    out6 = out1d.reshape(_L, OUT_DIM // 8, _B // 128, 8, 128)
    return out6.transpose(2, 4, 0, 1, 3).reshape(_B, _L, OUT_DIM)


# speedup vs baseline: 4.6334x; 1.3181x over previous
import functools

import jax
import jax.numpy as jnp
from jax import lax
from jax.experimental import pallas as pl
from jax.experimental.pallas import tpu as pltpu
from jax.experimental.pallas import tpu_sc as plsc

NUM_EMBEDDINGS = 100000
EMBEDDING_DIM = 128
OUT_DIM = 32

ROW_BLOCK = 4000


def _proj_body(table_ref, w_ref, b_ref, out_ref):
    t4 = table_ref[...].reshape(ROW_BLOCK // 4, 4, EMBEDDING_DIM)
    for k in range(4):
        acc = lax.dot_general(
            t4[:, k, :], w_ref[...],
            dimension_numbers=(((1,), (1,)), ((), ())),
            preferred_element_type=jnp.float32,
        )
        out_ref[:, k * OUT_DIM:(k + 1) * OUT_DIM] = acc + b_ref[...]


def _project_table(table, W, b2d):
    grid = NUM_EMBEDDINGS // ROW_BLOCK
    return pl.pallas_call(
        _proj_body,
        grid=(grid,),
        in_specs=[
            pl.BlockSpec((ROW_BLOCK, EMBEDDING_DIM), lambda i: (i, 0)),
            pl.BlockSpec((OUT_DIM, EMBEDDING_DIM), lambda i: (0, 0)),
            pl.BlockSpec((1, OUT_DIM), lambda i: (0, 0)),
        ],
        out_specs=pl.BlockSpec((ROW_BLOCK // 4, 4 * OUT_DIM), lambda i: (i, 0)),
        out_shape=jax.ShapeDtypeStruct((NUM_EMBEDDINGS // 4, 4 * OUT_DIM), jnp.float32),
    )(table, W, b2d)


_INFO = plsc.get_sparse_core_info()
_NC = _INFO.num_cores
_NS = _INFO.num_subcores
_NW = _NC * _NS

_B = 4096
_L = 50
_NB = _B // _NW


def _make_lookup():
    mesh = plsc.VectorSubcoreMesh(core_axis_name="c", subcore_axis_name="s")

    @functools.partial(
        pl.kernel,
        mesh=mesh,
        out_type=jax.ShapeDtypeStruct((_L * (OUT_DIM // 8) * (_B // 128) * 8 * 128,),
                                      jnp.float32),
        scratch_types=[
            pltpu.VMEM((_NB * _L,), jnp.int32),
            pltpu.VMEM((_NB,), jnp.int32),
            pltpu.VMEM((_NB,), jnp.int32),
            pltpu.VMEM((_NB, OUT_DIM), jnp.float32),
            pltpu.VMEM((_NB, OUT_DIM), jnp.float32),
            pltpu.VMEM((4096,), jnp.float32),
            pltpu.VMEM((4096,), jnp.float32),
            pltpu.SemaphoreType.DMA,
            pltpu.SemaphoreType.DMA,
            pltpu.SemaphoreType.DMA,
            pltpu.SemaphoreType.DMA,
        ],
        compiler_params=pltpu.CompilerParams(
            use_tc_tiling_on_sc=False, needs_layout_passes=False
        ),
    )
    def lookup_k(idx_hbm, proj_hbm, out_hbm, idx_v,
                 col0, col1, rows0, rows1, tile0, tile1,
                 gsem0, gsem1, wsem0, wsem1):
        cols, rows, tiles = [col0, col1], [rows0, rows1], [tile0, tile1]
        gsems, wsems = [gsem0, gsem1], [wsem0, wsem1]
        wid = lax.axis_index("s") * _NC + lax.axis_index("c")
        iota16 = lax.iota(jnp.int32, 16)
        iota128 = iota16 * 128
        iota50 = iota16 * _L

        pltpu.sync_copy(idx_hbm.at[pl.ds(wid * (_NB * _L), _NB * _L)], idx_v)

        def build_col(l, col):
            for k in range(8):
                vals = plsc.load_gather(idx_v, [iota50 + (16 * _L * k + l)])
                col[pl.ds(16 * k, 16)] = vals

        def transpose(rows_v, tile_v):
            def bbody(bt, c2):
                for j in range(8):
                    bi = bt * 8 + j
                    v0 = rows_v[bi, pl.ds(0, 16)]
                    v1 = rows_v[bi, pl.ds(16, 16)]
                    plsc.store_scatter(tile_v, [iota128 + bi], v0)
                    plsc.store_scatter(tile_v, [iota128 + (bi + 2048)], v1)
                return c2
            lax.fori_loop(0, _NB // 8, bbody, 0, unroll=False)

        build_col(0, cols[0])
        pltpu.async_copy(proj_hbm.at[cols[0]], rows[0], gsems[0])

        def pair(i, carry):
            for p in range(2):
                l = i * 2 + p
                cur, nxt = p, 1 - p

                @pl.when(l < _L - 1)
                def _():
                    build_col(l + 1, cols[nxt])
                    pltpu.async_copy(proj_hbm.at[cols[nxt]], rows[nxt], gsems[nxt])

                pltpu.make_async_copy(
                    proj_hbm.at[cols[cur]], rows[cur], gsems[cur]
                ).wait()

                @pl.when(i > 0)
                def _():
                    pltpu.make_async_copy(
                        tiles[cur], out_hbm.at[pl.ds(0, 4096)], wsems[cur]
                    ).wait()

                transpose(rows[cur], tiles[cur])
                for ot in range(4):
                    off = ((l * 4 + ot) * 32 + wid) * 1024
                    pltpu.async_copy(
                        tiles[cur].at[pl.ds(ot * 1024, 1024)],
                        out_hbm.at[pl.ds(off, 1024)],
                        wsems[cur],
                    )
            return carry

        lax.fori_loop(0, _L // 2, pair, 0, unroll=False)
        pltpu.make_async_copy(tiles[0], out_hbm.at[pl.ds(0, 4096)], wsems[0]).wait()
        pltpu.make_async_copy(tiles[1], out_hbm.at[pl.ds(0, 4096)], wsems[1]).wait()

    return lookup_k


def kernel(x, table, W, b):
    proj = _project_table(table, W, b.reshape(1, OUT_DIM))
    proj = proj.reshape(NUM_EMBEDDINGS, OUT_DIM)
    out1d = _make_lookup()(x.reshape(_B * _L), proj)
    out6 = out1d.reshape(_L, OUT_DIM // 8, _B // 128, 8, 128)
    return out6.transpose(2, 4, 0, 1, 3).reshape(_B, _L, OUT_DIM)
